# baseline (device time: 31453 ns/iter reference)
import jax
import jax.numpy as jnp
from jax import lax
from jax.experimental import pallas as pl
from jax.experimental.pallas import tpu as pltpu

N_DEV = 16
P = 4
Z = 4
N_DIR = 2
GA = 3
COLW = 128
A_N = N_DIR * GA * COLW


def kernel(x, w_mat):
    m, k = x.shape
    _, n = w_mat.shape
    m_chunk = m // N_DEV
    blk_rows = m // P
    b_n = n - A_N

    def body(x_ref, w_ref, out_ref, xp_ref, pacc_ref, bacc_ref,
             a1_buf, a2_buf, a_r, b1_buf, b2_buf, b_r,
             a1_send, a1_recv, a2_send, a2_recv,
             b1_send, b1_recv, b2_send, b2_recv):
        my = lax.axis_index("i")
        q = lax.rem(my, P)
        p = lax.div(my, P)
        plane_r = p * P + lax.rem(q + 1, P)
        plane_l = p * P + lax.rem(q + 3, P)
        col_u = lax.rem(p + 1, Z) * P + q
        col_d = lax.rem(p + 3, Z) * P + q

        barrier_sem = pltpu.get_barrier_semaphore()
        for nbr in (plane_l, plane_r, col_u, col_d):
            pl.semaphore_signal(
                barrier_sem, inc=1,
                device_id=(nbr,), device_id_type=pl.DeviceIdType.MESH,
            )

        for qb in range(P):
            for t in range(Z):
                xp_ref[qb * blk_rows + t * m_chunk:
                       qb * blk_rows + (t + 1) * m_chunk, :] = (
                    x_ref[(Z * t + qb) * m_chunk:
                          (Z * t + qb + 1) * m_chunk, :]
                )

        def compute_qblock(qb):
            pacc_ref[pl.ds(qb * blk_rows, blk_rows), :] = jnp.dot(
                xp_ref[pl.ds(qb * blk_rows, blk_rows), :],
                w_ref[:, 0:A_N], preferred_element_type=jnp.float32,
            )

        def compute_bblock(t):
            bacc_ref[pl.ds(t * blk_rows, blk_rows), :] = jnp.dot(
                x_ref[pl.ds(t * blk_rows, blk_rows), :],
                w_ref[:, A_N:n], preferred_element_type=jnp.float32,
            )

        compute_bblock(lax.rem(p + 3, Z))
        compute_bblock(lax.rem(p + 1, Z))
        compute_qblock(lax.rem(q + 3, P))
        compute_qblock(lax.rem(q + 1, P))

        pl.semaphore_wait(barrier_sem, 4)

        sa = [(d, g) for d in range(N_DIR) for g in range(GA)]
        sb = list(range(N_DIR))

        def acol0(d, g):
            return (d * GA + g) * COLW

        def bcol0(d):
            return A_N + d * COLW

        def qblock(qb, d, g):
            return pacc_ref[pl.ds(qb * blk_rows, blk_rows),
                            acol0(d, g):acol0(d, g) + COLW]

        def bblock(t, d):
            return bacc_ref[pl.ds(t * blk_rows, blk_rows),
                            d * COLW:(d + 1) * COLW]

        def a_group(d, g, t):
            return a_r[d, g, pl.ds(t * m_chunk, m_chunk), :]

        def b_group(d, j):
            return b_r[d, pl.ds(j * m_chunk, m_chunk), :]

        def plane_send_idx(d, s):
            return lax.rem(q + 3 - s, P) if d == 0 else lax.rem(q + s + 1, P)

        def col_send_idx(d, s):
            return lax.rem(p + 3 - s, Z) if d == 0 else lax.rem(p + s + 1, Z)

        def make_rdma(buf, send, recv, idx, s, to_dev):
            return pltpu.make_async_remote_copy(
                src_ref=buf.at[idx + (s,)],
                dst_ref=buf.at[idx + (s + 1,)],
                send_sem=send.at[idx + (s,)],
                recv_sem=recv.at[idx + (s,)],
                device_id=(to_dev,),
                device_id_type=pl.DeviceIdType.MESH,
            )

        plane_to = {0: plane_r, 1: plane_l}
        col_to = {0: col_u, 1: col_d}
        rdmas = {}

        def start(key, rdma):
            rdmas[key] = rdma
            rdma.start()

        for d in sb:
            b1_buf[d, 0, :, :] = bblock(col_send_idx(d, 0), d)
            start(("b1", d, 0),
                  make_rdma(b1_buf, b1_send, b1_recv, (d,), 0, col_to[d]))
        for d, g in sa:
            a1_buf[d, g, 0, :, :] = qblock(plane_send_idx(d, 0), d, g)
            start(("a1", d, g, 0),
                  make_rdma(a1_buf, a1_send, a1_recv, (d, g), 0, plane_to[d]))

        compute_bblock(lax.rem(p + 2, Z))
        compute_bblock(p)
        compute_qblock(lax.rem(q + 2, P))
        compute_qblock(q)

        for s in range(1, 3):
            for d, g in sa:
                rdmas[("a1", d, g, s - 1)].wait_recv()
                a1_buf[d, g, s, :, :] = (
                    a1_buf[d, g, s, :, :] + qblock(plane_send_idx(d, s), d, g)
                )
                start(("a1", d, g, s),
                      make_rdma(a1_buf, a1_send, a1_recv, (d, g), s,
                                plane_to[d]))
            for d in sb:
                rdmas[("b1", d, s - 1)].wait_recv()
                b1_buf[d, s, :, :] = (
                    b1_buf[d, s, :, :] + bblock(col_send_idx(d, s), d)
                )
                start(("b1", d, s),
                      make_rdma(b1_buf, b1_send, b1_recv, (d,), s, col_to[d]))

        for d, g in sa:
            rdmas[("a1", d, g, 2)].wait_recv()
            t0 = col_send_idx(d, 0)
            a2_buf[d, g, 0, :, :] = (
                a1_buf[d, g, 3, pl.ds(t0 * m_chunk, m_chunk), :]
                + pacc_ref[pl.ds(q * blk_rows + t0 * m_chunk, m_chunk),
                           acol0(d, g):acol0(d, g) + COLW]
            )
            start(("a2", d, g, 0),
                  make_rdma(a2_buf, a2_send, a2_recv, (d, g), 0, col_to[d]))
        for d in sb:
            rdmas[("b1", d, 2)].wait_recv()
            j0 = plane_send_idx(d, 0)
            b2_buf[d, 0, :, :] = (
                b1_buf[d, 3, pl.ds(j0 * m_chunk, m_chunk), :]
                + bacc_ref[pl.ds(p * blk_rows + j0 * m_chunk, m_chunk),
                           d * COLW:(d + 1) * COLW]
            )
            start(("b2", d, 0),
                  make_rdma(b2_buf, b2_send, b2_recv, (d,), 0, plane_to[d]))
        for d, g in sa:
            a_r[d, g, :, :] = a1_buf[d, g, 3, :, :] + qblock(q, d, g)
        for d in sb:
            b_r[d, :, :] = b1_buf[d, 3, :, :] + bblock(p, d)

        for s in range(1, 3):
            for d, g in sa:
                rdmas[("a2", d, g, s - 1)].wait_recv()
                a2_buf[d, g, s, :, :] = (
                    a2_buf[d, g, s, :, :] + a_group(d, g, col_send_idx(d, s))
                )
                start(("a2", d, g, s),
                      make_rdma(a2_buf, a2_send, a2_recv, (d, g), s,
                                col_to[d]))
            for d in sb:
                rdmas[("b2", d, s - 1)].wait_recv()
                b2_buf[d, s, :, :] = (
                    b2_buf[d, s, :, :] + b_group(d, plane_send_idx(d, s))
                )
                start(("b2", d, s),
                      make_rdma(b2_buf, b2_send, b2_recv, (d,), s,
                                plane_to[d]))

        for d in sb:
            rdmas[("b2", d, 2)].wait_recv()
            out_ref[:, bcol0(d):bcol0(d) + COLW] = jnp.maximum(
                b2_buf[d, 3, :, :] + b_group(d, q), 0.0
            )
        for d, g in sa:
            rdmas[("a2", d, g, 2)].wait_recv()
            out_ref[:, acol0(d, g):acol0(d, g) + COLW] = jnp.maximum(
                a2_buf[d, g, 3, :, :] + a_group(d, g, p), 0.0
            )

        for rdma in rdmas.values():
            rdma.wait_send()

    dma3 = lambda *shape: pltpu.SemaphoreType.DMA(shape)
    return pl.pallas_call(
        body,
        out_shape=jax.ShapeDtypeStruct((m_chunk, n), jnp.float32),
        in_specs=[
            pl.BlockSpec(memory_space=pltpu.VMEM),
            pl.BlockSpec(memory_space=pltpu.VMEM),
        ],
        out_specs=pl.BlockSpec(memory_space=pltpu.VMEM),
        scratch_shapes=[
            pltpu.VMEM((m, k), jnp.float32),
            pltpu.VMEM((m, A_N), jnp.float32),
            pltpu.VMEM((m, b_n), jnp.float32),
            pltpu.VMEM((N_DIR, GA, P, blk_rows, COLW), jnp.float32),
            pltpu.VMEM((N_DIR, GA, Z, m_chunk, COLW), jnp.float32),
            pltpu.VMEM((N_DIR, GA, blk_rows, COLW), jnp.float32),
            pltpu.VMEM((N_DIR, Z, blk_rows, COLW), jnp.float32),
            pltpu.VMEM((N_DIR, P, m_chunk, COLW), jnp.float32),
            pltpu.VMEM((N_DIR, blk_rows, COLW), jnp.float32),
            dma3(N_DIR, GA, P - 1), dma3(N_DIR, GA, P - 1),
            dma3(N_DIR, GA, Z - 1), dma3(N_DIR, GA, Z - 1),
            dma3(N_DIR, Z - 1), dma3(N_DIR, Z - 1),
            dma3(N_DIR, P - 1), dma3(N_DIR, P - 1),
        ],
        compiler_params=pltpu.CompilerParams(collective_id=0),
    )(x, w_mat)


# device time: 23517 ns/iter; 1.3375x vs baseline; 1.3375x over previous
import jax
import jax.numpy as jnp
from jax import lax
from jax.experimental import pallas as pl
from jax.experimental.pallas import tpu as pltpu

N_DEV = 16
P = 4
Z = 4
N_DIR = 2
GA = 3
COLW = 128
A_N = N_DIR * GA * COLW


def kernel(x, w_mat):
    m, k = x.shape
    _, n = w_mat.shape
    m_chunk = m // N_DEV
    blk_rows = m // P
    b_n = n - A_N

    def body(x_ref, w_ref, out_ref, xp_ref, pacc_ref, bacc_ref,
             a1_buf, a2_buf, a_r, b1_buf, b2_buf, b_r,
             a1_send, a1_recv, a2_send, a2_recv,
             b1_send, b1_recv, b2_send, b2_recv):
        my = lax.axis_index("i")
        q = lax.rem(my, P)
        p = lax.div(my, P)
        plane_r = p * P + lax.rem(q + 1, P)
        plane_l = p * P + lax.rem(q + 3, P)
        col_u = lax.rem(p + 1, Z) * P + q
        col_d = lax.rem(p + 3, Z) * P + q

        barrier_sem = pltpu.get_barrier_semaphore()
        for nbr in (plane_l, plane_r, col_u, col_d):
            pl.semaphore_signal(
                barrier_sem, inc=1,
                device_id=(nbr,), device_id_type=pl.DeviceIdType.MESH,
            )

        for qb in range(P):
            for t in range(Z):
                xp_ref[qb * blk_rows + t * m_chunk:
                       qb * blk_rows + (t + 1) * m_chunk, :] = (
                    x_ref[(Z * t + qb) * m_chunk:
                          (Z * t + qb + 1) * m_chunk, :]
                )

        def compute_qblock(qb):
            pacc_ref[pl.ds(qb * blk_rows, blk_rows), :] = jnp.dot(
                xp_ref[pl.ds(qb * blk_rows, blk_rows), :],
                w_ref[:, 0:A_N], preferred_element_type=jnp.float32,
            )

        def compute_bblock(t):
            bacc_ref[pl.ds(t * blk_rows, blk_rows), :] = jnp.dot(
                x_ref[pl.ds(t * blk_rows, blk_rows), :],
                w_ref[:, A_N:n], preferred_element_type=jnp.float32,
            )

        compute_bblock(lax.rem(p + 3, Z))
        compute_bblock(lax.rem(p + 1, Z))
        compute_qblock(lax.rem(q + 3, P))
        compute_qblock(lax.rem(q + 1, P))

        pl.semaphore_wait(barrier_sem, 4)

        sa = [(d, g) for d in range(N_DIR) for g in range(GA)]
        sb = list(range(N_DIR))

        def acol0(d, g):
            return (d * GA + g) * COLW

        def bcol0(d):
            return A_N + d * COLW

        def qblock(qb, d, g):
            return pacc_ref[pl.ds(qb * blk_rows, blk_rows),
                            acol0(d, g):acol0(d, g) + COLW]

        def bblock(t, d):
            return bacc_ref[pl.ds(t * blk_rows, blk_rows),
                            d * COLW:(d + 1) * COLW]

        def a_group(d, g, t):
            return a_r[d, g, pl.ds(t * m_chunk, m_chunk), :]

        def b_group(d, j):
            return b_r[d, pl.ds(j * m_chunk, m_chunk), :]

        def plane_send_idx(d, s):
            return lax.rem(q + 3 - s, P) if d == 0 else lax.rem(q + s + 1, P)

        def col_send_idx(d, s):
            return lax.rem(p + 3 - s, Z) if d == 0 else lax.rem(p + s + 1, Z)

        def make_rdma(buf, send, recv, idx, s, to_dev):
            return pltpu.make_async_remote_copy(
                src_ref=buf.at[idx + (s,)],
                dst_ref=buf.at[idx + (s + 1,)],
                send_sem=send.at[idx + (s,)],
                recv_sem=recv.at[idx + (s,)],
                device_id=(to_dev,),
                device_id_type=pl.DeviceIdType.MESH,
            )

        plane_to = {0: plane_r, 1: plane_l}
        col_to = {0: col_u, 1: col_d}
        rdmas = {}

        def start(key, rdma):
            rdmas[key] = rdma
            rdma.start()

        for d in sb:
            b1_buf[d, 0, :, :] = bblock(col_send_idx(d, 0), d).astype(jnp.bfloat16)
            start(("b1", d, 0),
                  make_rdma(b1_buf, b1_send, b1_recv, (d,), 0, col_to[d]))
        for d, g in sa:
            a1_buf[d, g, 0, :, :] = qblock(plane_send_idx(d, 0), d, g).astype(jnp.bfloat16)
            start(("a1", d, g, 0),
                  make_rdma(a1_buf, a1_send, a1_recv, (d, g), 0, plane_to[d]))

        compute_bblock(lax.rem(p + 2, Z))
        compute_bblock(p)
        compute_qblock(lax.rem(q + 2, P))
        compute_qblock(q)

        for s in range(1, 3):
            for d, g in sa:
                rdmas[("a1", d, g, s - 1)].wait_recv()
                a1_buf[d, g, s, :, :] = (
                    a1_buf[d, g, s, :, :].astype(jnp.float32)
                    + qblock(plane_send_idx(d, s), d, g)
                ).astype(jnp.bfloat16)
                start(("a1", d, g, s),
                      make_rdma(a1_buf, a1_send, a1_recv, (d, g), s,
                                plane_to[d]))
            for d in sb:
                rdmas[("b1", d, s - 1)].wait_recv()
                b1_buf[d, s, :, :] = (
                    b1_buf[d, s, :, :].astype(jnp.float32)
                    + bblock(col_send_idx(d, s), d)
                ).astype(jnp.bfloat16)
                start(("b1", d, s),
                      make_rdma(b1_buf, b1_send, b1_recv, (d,), s, col_to[d]))

        for d, g in sa:
            rdmas[("a1", d, g, 2)].wait_recv()
            t0 = col_send_idx(d, 0)
            a2_buf[d, g, 0, :, :] = (
                a1_buf[d, g, 3, pl.ds(t0 * m_chunk, m_chunk), :]
                .astype(jnp.float32)
                + pacc_ref[pl.ds(q * blk_rows + t0 * m_chunk, m_chunk),
                           acol0(d, g):acol0(d, g) + COLW]
            ).astype(jnp.bfloat16)
            start(("a2", d, g, 0),
                  make_rdma(a2_buf, a2_send, a2_recv, (d, g), 0, col_to[d]))
        for d in sb:
            rdmas[("b1", d, 2)].wait_recv()
            j0 = plane_send_idx(d, 0)
            b2_buf[d, 0, :, :] = (
                b1_buf[d, 3, pl.ds(j0 * m_chunk, m_chunk), :]
                .astype(jnp.float32)
                + bacc_ref[pl.ds(p * blk_rows + j0 * m_chunk, m_chunk),
                           d * COLW:(d + 1) * COLW]
            ).astype(jnp.bfloat16)
            start(("b2", d, 0),
                  make_rdma(b2_buf, b2_send, b2_recv, (d,), 0, plane_to[d]))
        for d, g in sa:
            a_r[d, g, :, :] = (a1_buf[d, g, 3, :, :].astype(jnp.float32)
                               + qblock(q, d, g))
        for d in sb:
            b_r[d, :, :] = b1_buf[d, 3, :, :].astype(jnp.float32) + bblock(p, d)

        for s in range(1, 3):
            for d, g in sa:
                rdmas[("a2", d, g, s - 1)].wait_recv()
                a2_buf[d, g, s, :, :] = (
                    a2_buf[d, g, s, :, :].astype(jnp.float32)
                    + a_group(d, g, col_send_idx(d, s))
                ).astype(jnp.bfloat16)
                start(("a2", d, g, s),
                      make_rdma(a2_buf, a2_send, a2_recv, (d, g), s,
                                col_to[d]))
            for d in sb:
                rdmas[("b2", d, s - 1)].wait_recv()
                b2_buf[d, s, :, :] = (
                    b2_buf[d, s, :, :].astype(jnp.float32)
                    + b_group(d, plane_send_idx(d, s))
                ).astype(jnp.bfloat16)
                start(("b2", d, s),
                      make_rdma(b2_buf, b2_send, b2_recv, (d,), s,
                                plane_to[d]))

        for d in sb:
            rdmas[("b2", d, 2)].wait_recv()
            out_ref[:, bcol0(d):bcol0(d) + COLW] = jnp.maximum(
                b2_buf[d, 3, :, :].astype(jnp.float32) + b_group(d, q), 0.0
            )
        for d, g in sa:
            rdmas[("a2", d, g, 2)].wait_recv()
            out_ref[:, acol0(d, g):acol0(d, g) + COLW] = jnp.maximum(
                a2_buf[d, g, 3, :, :].astype(jnp.float32) + a_group(d, g, p),
                0.0
            )

        for rdma in rdmas.values():
            rdma.wait_send()

    dma3 = lambda *shape: pltpu.SemaphoreType.DMA(shape)
    return pl.pallas_call(
        body,
        out_shape=jax.ShapeDtypeStruct((m_chunk, n), jnp.float32),
        in_specs=[
            pl.BlockSpec(memory_space=pltpu.VMEM),
            pl.BlockSpec(memory_space=pltpu.VMEM),
        ],
        out_specs=pl.BlockSpec(memory_space=pltpu.VMEM),
        scratch_shapes=[
            pltpu.VMEM((m, k), jnp.float32),
            pltpu.VMEM((m, A_N), jnp.float32),
            pltpu.VMEM((m, b_n), jnp.float32),
            pltpu.VMEM((N_DIR, GA, P, blk_rows, COLW), jnp.bfloat16),
            pltpu.VMEM((N_DIR, GA, Z, m_chunk, COLW), jnp.bfloat16),
            pltpu.VMEM((N_DIR, GA, blk_rows, COLW), jnp.float32),
            pltpu.VMEM((N_DIR, Z, blk_rows, COLW), jnp.bfloat16),
            pltpu.VMEM((N_DIR, P, m_chunk, COLW), jnp.bfloat16),
            pltpu.VMEM((N_DIR, blk_rows, COLW), jnp.float32),
            dma3(N_DIR, GA, P - 1), dma3(N_DIR, GA, P - 1),
            dma3(N_DIR, GA, Z - 1), dma3(N_DIR, GA, Z - 1),
            dma3(N_DIR, Z - 1), dma3(N_DIR, Z - 1),
            dma3(N_DIR, P - 1), dma3(N_DIR, P - 1),
        ],
        compiler_params=pltpu.CompilerParams(collective_id=0),
    )(x, w_mat)


# device time: 21647 ns/iter; 1.4530x vs baseline; 1.0864x over previous
import jax
import jax.numpy as jnp
from jax import lax
from jax.experimental import pallas as pl
from jax.experimental.pallas import tpu as pltpu

N_DEV = 16
NA = 6
NB = 2
COLW = 128
A_N = NA * COLW
BF = jnp.bfloat16
F32 = jnp.float32


def kernel(x, w_mat):
    m, k = x.shape
    _, n = w_mat.shape
    m_chunk = m // N_DEV
    blk = m // 4

    def body(x_ref, w_ref, out_ref, xp_ref, pacc_ref, bacc_ref,
             a1_s1, a1_s2, a_r, a2_s1, a2_s2,
             b1_s1, b1_s2, b_r, b2_s1, b2_s2,
             a1s1_snd, a1s1_rcv, a1s2_snd, a1s2_rcv,
             a2s1_snd, a2s1_rcv, a2s2_snd, a2s2_rcv,
             b1s1_snd, b1s1_rcv, b1s2_snd, b1s2_rcv,
             b2s1_snd, b2s1_rcv, b2s2_snd, b2s2_rcv):
        my = lax.axis_index("i")
        q = lax.rem(my, 4)
        p = lax.div(my, 4)
        qh = lax.div(q, 2)
        ql = lax.rem(q, 2)
        ph = lax.div(p, 2)
        pll = lax.rem(p, 2)
        sy = ql ^ qh

        dev_plane_x = p * 4 + (q ^ 1)
        dev_plane_y = p * 4 + (3 - q)
        dev_col_1 = (p ^ 1) * 4 + q
        dev_col_2 = (p ^ 2) * 4 + q

        PX = dict(p1=dev_plane_x, lo=1 - sy, hi=2 + sy,
                  pos_own=qh, mate=3 - q, pos_mate=1 - qh, p2=dev_plane_y)
        PY = dict(p1=dev_plane_y, lo=2 - 2 * qh, hi=3 - 2 * qh,
                  pos_own=ql, mate=q ^ 1, pos_mate=1 - ql, p2=dev_plane_x)
        PZ = dict(p1=dev_col_1, lo=1 - pll, hi=3 - pll,
                  pos_own=ph, mate=p ^ 2, pos_mate=1 - ph, p2=dev_col_2)

        def fam_a(i):
            return PX if i % 2 == 0 else PY

        barrier_sem = pltpu.get_barrier_semaphore()
        for nbr in (dev_plane_x, dev_plane_y, dev_col_1, dev_col_2):
            pl.semaphore_signal(
                barrier_sem, inc=1,
                device_id=(nbr,), device_id_type=pl.DeviceIdType.MESH,
            )

        for qb in range(4):
            for t in range(4):
                xp_ref[qb * blk + t * m_chunk:
                       qb * blk + (t + 1) * m_chunk, :] = (
                    x_ref[(4 * t + qb) * m_chunk:
                          (4 * t + qb + 1) * m_chunk, :]
                )

        bacc_ref[...] = jnp.dot(x_ref[...], w_ref[:, A_N:n],
                                preferred_element_type=F32)
        pacc_ref[0:2 * blk, :] = jnp.dot(xp_ref[0:2 * blk, :],
                                         w_ref[:, 0:A_N],
                                         preferred_element_type=F32)
        pacc_ref[2 * blk:m, :] = jnp.dot(xp_ref[2 * blk:m, :],
                                         w_ref[:, 0:A_N],
                                         preferred_element_type=F32)

        pl.semaphore_wait(barrier_sem, 4)

        def qblock(qb, i):
            return pacc_ref[pl.ds(qb * blk, blk), i * COLW:(i + 1) * COLW]

        def bblock(t, i):
            return bacc_ref[pl.ds(t * blk, blk), i * COLW:(i + 1) * COLW]

        rdmas = {}

        def xchg(key, buf, i, snd, rcv, peer):
            r = pltpu.make_async_remote_copy(
                src_ref=buf.at[i, 0],
                dst_ref=buf.at[i, 1],
                send_sem=snd.at[i],
                recv_sem=rcv.at[i],
                device_id=(peer,),
                device_id_type=pl.DeviceIdType.MESH,
            )
            rdmas[key] = r
            r.start()

        for i in range(NB):
            b1_s1[i, 0, 0:blk, :] = bblock(PZ["lo"], i).astype(BF)
            b1_s1[i, 0, blk:2 * blk, :] = bblock(PZ["hi"], i).astype(BF)
            xchg(("b1s1", i), b1_s1, i, b1s1_snd, b1s1_rcv, PZ["p1"])
        for i in range(NA):
            f = fam_a(i)
            a1_s1[i, 0, 0:blk, :] = qblock(f["lo"], i).astype(BF)
            a1_s1[i, 0, blk:2 * blk, :] = qblock(f["hi"], i).astype(BF)
            xchg(("a1s1", i), a1_s1, i, a1s1_snd, a1s1_rcv, f["p1"])

        for i in range(NA):
            f = fam_a(i)
            rdmas[("a1s1", i)].wait_recv()
            a1_s2[i, 0, :, :] = (
                a1_s1[i, 1, pl.ds(f["pos_mate"] * blk, blk), :].astype(F32)
                + qblock(f["mate"], i)
            ).astype(BF)
            xchg(("a1s2", i), a1_s2, i, a1s2_snd, a1s2_rcv, f["p2"])
            a_r[i, :, :] = (
                a1_s1[i, 1, pl.ds(f["pos_own"] * blk, blk), :].astype(F32)
                + qblock(q, i)
            )
        for i in range(NB):
            rdmas[("b1s1", i)].wait_recv()
            b1_s2[i, 0, :, :] = (
                b1_s1[i, 1, pl.ds(PZ["pos_mate"] * blk, blk), :].astype(F32)
                + bblock(PZ["mate"], i)
            ).astype(BF)
            xchg(("b1s2", i), b1_s2, i, b1s2_snd, b1s2_rcv, PZ["p2"])
            b_r[i, :, :] = (
                b1_s1[i, 1, pl.ds(PZ["pos_own"] * blk, blk), :].astype(F32)
                + bblock(p, i)
            )

        def a_group(i, t):
            return a_r[i, pl.ds(t * m_chunk, m_chunk), :]

        def b_group(i, j):
            return b_r[i, pl.ds(j * m_chunk, m_chunk), :]

        for i in range(NA):
            rdmas[("a1s2", i)].wait_recv()
            a_r[i, :, :] = a_r[i, :, :] + a1_s2[i, 1, :, :].astype(F32)
            a2_s1[i, 0, 0:m_chunk, :] = a_group(i, PZ["lo"]).astype(BF)
            a2_s1[i, 0, m_chunk:2 * m_chunk, :] = (
                a_group(i, PZ["hi"]).astype(BF))
            xchg(("a2s1", i), a2_s1, i, a2s1_snd, a2s1_rcv, PZ["p1"])
        for i in range(NB):
            rdmas[("b1s2", i)].wait_recv()
            b_r[i, :, :] = b_r[i, :, :] + b1_s2[i, 1, :, :].astype(F32)
            f = fam_a(i)
            b2_s1[i, 0, 0:m_chunk, :] = b_group(i, f["lo"]).astype(BF)
            b2_s1[i, 0, m_chunk:2 * m_chunk, :] = (
                b_group(i, f["hi"]).astype(BF))
            xchg(("b2s1", i), b2_s1, i, b2s1_snd, b2s1_rcv, f["p1"])

        for i in range(NA):
            rdmas[("a2s1", i)].wait_recv()
            a2_s2[i, 0, :, :] = (
                a2_s1[i, 1, pl.ds(PZ["pos_mate"] * m_chunk, m_chunk), :]
                .astype(F32) + a_group(i, PZ["mate"])
            ).astype(BF)
            xchg(("a2s2", i), a2_s2, i, a2s2_snd, a2s2_rcv, PZ["p2"])
        for i in range(NB):
            rdmas[("b2s1", i)].wait_recv()
            f = fam_a(i)
            b2_s2[i, 0, :, :] = (
                b2_s1[i, 1, pl.ds(f["pos_mate"] * m_chunk, m_chunk), :]
                .astype(F32) + b_group(i, f["mate"])
            ).astype(BF)
            xchg(("b2s2", i), b2_s2, i, b2s2_snd, b2s2_rcv, f["p2"])

        for i in range(NA):
            rdmas[("a2s2", i)].wait_recv()
            out_ref[:, i * COLW:(i + 1) * COLW] = jnp.maximum(
                a2_s1[i, 1, pl.ds(PZ["pos_own"] * m_chunk, m_chunk), :]
                .astype(F32) + a_group(i, p)
                + a2_s2[i, 1, :, :].astype(F32),
                0.0,
            )
        for i in range(NB):
            rdmas[("b2s2", i)].wait_recv()
            f = fam_a(i)
            out_ref[:, A_N + i * COLW:A_N + (i + 1) * COLW] = jnp.maximum(
                b2_s1[i, 1, pl.ds(f["pos_own"] * m_chunk, m_chunk), :]
                .astype(F32) + b_group(i, q)
                + b2_s2[i, 1, :, :].astype(F32),
                0.0,
            )

        for rdma in rdmas.values():
            rdma.wait_send()

    dma = pltpu.SemaphoreType.DMA
    return pl.pallas_call(
        body,
        out_shape=jax.ShapeDtypeStruct((m_chunk, n), F32),
        in_specs=[
            pl.BlockSpec(memory_space=pltpu.VMEM),
            pl.BlockSpec(memory_space=pltpu.VMEM),
        ],
        out_specs=pl.BlockSpec(memory_space=pltpu.VMEM),
        scratch_shapes=[
            pltpu.VMEM((m, k), F32),
            pltpu.VMEM((m, A_N), F32),
            pltpu.VMEM((m, n - A_N), F32),
            pltpu.VMEM((NA, 2, 2 * blk, COLW), BF),
            pltpu.VMEM((NA, 2, blk, COLW), BF),
            pltpu.VMEM((NA, blk, COLW), F32),
            pltpu.VMEM((NA, 2, 2 * m_chunk, COLW), BF),
            pltpu.VMEM((NA, 2, m_chunk, COLW), BF),
            pltpu.VMEM((NB, 2, 2 * blk, COLW), BF),
            pltpu.VMEM((NB, 2, blk, COLW), BF),
            pltpu.VMEM((NB, blk, COLW), F32),
            pltpu.VMEM((NB, 2, 2 * m_chunk, COLW), BF),
            pltpu.VMEM((NB, 2, m_chunk, COLW), BF),
            dma((NA,)), dma((NA,)), dma((NA,)), dma((NA,)),
            dma((NA,)), dma((NA,)), dma((NA,)), dma((NA,)),
            dma((NB,)), dma((NB,)), dma((NB,)), dma((NB,)),
            dma((NB,)), dma((NB,)), dma((NB,)), dma((NB,)),
        ],
        compiler_params=pltpu.CompilerParams(collective_id=0),
    )(x, w_mat)


# device time: 20878 ns/iter; 1.5065x vs baseline; 1.0368x over previous
import jax
import jax.numpy as jnp
from jax import lax
from jax.experimental import pallas as pl
from jax.experimental.pallas import tpu as pltpu

N_DEV = 16
NA = 6
NB = 2
COLW = 128
A_N = NA * COLW
BF = jnp.bfloat16
F32 = jnp.float32


def kernel(x, w_mat):
    m, k = x.shape
    _, n = w_mat.shape
    m_chunk = m // N_DEV
    blk = m // 4

    def body(x_ref, w_ref, out_ref, xp_ref, pacc_ref, bacc_ref,
             a1_s1, a1_s2, a_r, a2_s1, a2_s2,
             b1_s1, b1_s2, b_r, b2_s1, b2_s2,
             a1s1_snd, a1s1_rcv, a1s2_snd, a1s2_rcv,
             a2s1_snd, a2s1_rcv, a2s2_snd, a2s2_rcv,
             b1s1_snd, b1s1_rcv, b1s2_snd, b1s2_rcv,
             b2s1_snd, b2s1_rcv, b2s2_snd, b2s2_rcv):
        my = lax.axis_index("i")
        q = lax.rem(my, 4)
        p = lax.div(my, 4)
        qh = lax.div(q, 2)
        ql = lax.rem(q, 2)
        ph = lax.div(p, 2)
        pll = lax.rem(p, 2)
        sy = ql ^ qh

        dev_plane_x = p * 4 + (q ^ 1)
        dev_plane_y = p * 4 + (3 - q)
        dev_col_1 = (p ^ 1) * 4 + q
        dev_col_2 = (p ^ 2) * 4 + q

        PX = dict(p1=dev_plane_x, lo=1 - sy, hi=2 + sy,
                  pos_own=qh, mate=3 - q, pos_mate=1 - qh, p2=dev_plane_y)
        PY = dict(p1=dev_plane_y, lo=2 - 2 * qh, hi=3 - 2 * qh,
                  pos_own=ql, mate=q ^ 1, pos_mate=1 - ql, p2=dev_plane_x)
        PZ = dict(p1=dev_col_1, lo=1 - pll, hi=3 - pll,
                  pos_own=ph, mate=p ^ 2, pos_mate=1 - ph, p2=dev_col_2)

        def fam_a(i):
            return PX if i % 2 == 0 else PY

        barrier_sem = pltpu.get_barrier_semaphore()
        for nbr in (dev_plane_x, dev_plane_y, dev_col_1, dev_col_2):
            pl.semaphore_signal(
                barrier_sem, inc=1,
                device_id=(nbr,), device_id_type=pl.DeviceIdType.MESH,
            )

        def qblock(qb, i):
            return pacc_ref[pl.ds(qb * blk, blk), i * COLW:(i + 1) * COLW]

        def bblock(t, i):
            return bacc_ref[pl.ds(t * blk, blk), i * COLW:(i + 1) * COLW]

        rdmas = {}

        def xchg(key, buf, i, snd, rcv, peer):
            r = pltpu.make_async_remote_copy(
                src_ref=buf.at[i, 0],
                dst_ref=buf.at[i, 1],
                send_sem=snd.at[i],
                recv_sem=rcv.at[i],
                device_id=(peer,),
                device_id_type=pl.DeviceIdType.MESH,
            )
            rdmas[key] = r
            r.start()

        bacc_ref[...] = jnp.dot(x_ref[...], w_ref[:, A_N:n],
                                preferred_element_type=F32)
        pl.semaphore_wait(barrier_sem, 4)
        for i in range(NB):
            b1_s1[i, 0, 0:blk, :] = bblock(PZ["lo"], i).astype(BF)
            b1_s1[i, 0, blk:2 * blk, :] = bblock(PZ["hi"], i).astype(BF)
            xchg(("b1s1", i), b1_s1, i, b1s1_snd, b1s1_rcv, PZ["p1"])

        for qb in range(4):
            for t in range(4):
                xp_ref[qb * blk + t * m_chunk:
                       qb * blk + (t + 1) * m_chunk, :] = (
                    x_ref[(4 * t + qb) * m_chunk:
                          (4 * t + qb + 1) * m_chunk, :]
                )
        pacc_ref[0:2 * blk, :] = jnp.dot(xp_ref[0:2 * blk, :],
                                         w_ref[:, 0:A_N],
                                         preferred_element_type=F32)
        pacc_ref[2 * blk:m, :] = jnp.dot(xp_ref[2 * blk:m, :],
                                         w_ref[:, 0:A_N],
                                         preferred_element_type=F32)
        for i in range(NA):
            f = fam_a(i)
            a1_s1[i, 0, 0:blk, :] = qblock(f["lo"], i).astype(BF)
            a1_s1[i, 0, blk:2 * blk, :] = qblock(f["hi"], i).astype(BF)
            xchg(("a1s1", i), a1_s1, i, a1s1_snd, a1s1_rcv, f["p1"])

        for i in range(NB):
            rdmas[("b1s1", i)].wait_recv()
            b1_s2[i, 0, :, :] = (
                b1_s1[i, 1, pl.ds(PZ["pos_mate"] * blk, blk), :].astype(F32)
                + bblock(PZ["mate"], i)
            ).astype(BF)
            xchg(("b1s2", i), b1_s2, i, b1s2_snd, b1s2_rcv, PZ["p2"])
        for i in range(NA):
            f = fam_a(i)
            rdmas[("a1s1", i)].wait_recv()
            a1_s2[i, 0, :, :] = (
                a1_s1[i, 1, pl.ds(f["pos_mate"] * blk, blk), :].astype(F32)
                + qblock(f["mate"], i)
            ).astype(BF)
            xchg(("a1s2", i), a1_s2, i, a1s2_snd, a1s2_rcv, f["p2"])
        for i in range(NB):
            b_r[i, :, :] = (
                b1_s1[i, 1, pl.ds(PZ["pos_own"] * blk, blk), :].astype(F32)
                + bblock(p, i)
            )
        for i in range(NA):
            f = fam_a(i)
            a_r[i, :, :] = (
                a1_s1[i, 1, pl.ds(f["pos_own"] * blk, blk), :].astype(F32)
                + qblock(q, i)
            )

        def a_group(i, t):
            return a_r[i, pl.ds(t * m_chunk, m_chunk), :]

        def b_group(i, j):
            return b_r[i, pl.ds(j * m_chunk, m_chunk), :]

        def a_s2r(i, t):
            return a1_s2[i, 1, pl.ds(t * m_chunk, m_chunk), :].astype(F32)

        def b_s2r(i, j):
            return b1_s2[i, 1, pl.ds(j * m_chunk, m_chunk), :].astype(F32)

        for i in range(NB):
            rdmas[("b1s2", i)].wait_recv()
            f = fam_a(i)
            b2_s1[i, 0, 0:m_chunk, :] = (
                b_group(i, f["lo"]) + b_s2r(i, f["lo"])).astype(BF)
            b2_s1[i, 0, m_chunk:2 * m_chunk, :] = (
                b_group(i, f["hi"]) + b_s2r(i, f["hi"])).astype(BF)
            xchg(("b2s1", i), b2_s1, i, b2s1_snd, b2s1_rcv, f["p1"])
        for i in range(NA):
            rdmas[("a1s2", i)].wait_recv()
            a2_s1[i, 0, 0:m_chunk, :] = (
                a_group(i, PZ["lo"]) + a_s2r(i, PZ["lo"])).astype(BF)
            a2_s1[i, 0, m_chunk:2 * m_chunk, :] = (
                a_group(i, PZ["hi"]) + a_s2r(i, PZ["hi"])).astype(BF)
            xchg(("a2s1", i), a2_s1, i, a2s1_snd, a2s1_rcv, PZ["p1"])
        for i in range(NB):
            b_r[i, :, :] = b_r[i, :, :] + b1_s2[i, 1, :, :].astype(F32)
        for i in range(NA):
            a_r[i, :, :] = a_r[i, :, :] + a1_s2[i, 1, :, :].astype(F32)

        for i in range(NA):
            rdmas[("a2s1", i)].wait_recv()
            a2_s2[i, 0, :, :] = (
                a2_s1[i, 1, pl.ds(PZ["pos_mate"] * m_chunk, m_chunk), :]
                .astype(F32) + a_group(i, PZ["mate"])
            ).astype(BF)
            xchg(("a2s2", i), a2_s2, i, a2s2_snd, a2s2_rcv, PZ["p2"])
        for i in range(NB):
            rdmas[("b2s1", i)].wait_recv()
            f = fam_a(i)
            b2_s2[i, 0, :, :] = (
                b2_s1[i, 1, pl.ds(f["pos_mate"] * m_chunk, m_chunk), :]
                .astype(F32) + b_group(i, f["mate"])
            ).astype(BF)
            xchg(("b2s2", i), b2_s2, i, b2s2_snd, b2s2_rcv, f["p2"])

        for i in range(NA):
            rdmas[("a2s2", i)].wait_recv()
            out_ref[:, i * COLW:(i + 1) * COLW] = jnp.maximum(
                a2_s1[i, 1, pl.ds(PZ["pos_own"] * m_chunk, m_chunk), :]
                .astype(F32) + a_group(i, p)
                + a2_s2[i, 1, :, :].astype(F32),
                0.0,
            )
        for i in range(NB):
            rdmas[("b2s2", i)].wait_recv()
            f = fam_a(i)
            out_ref[:, A_N + i * COLW:A_N + (i + 1) * COLW] = jnp.maximum(
                b2_s1[i, 1, pl.ds(f["pos_own"] * m_chunk, m_chunk), :]
                .astype(F32) + b_group(i, q)
                + b2_s2[i, 1, :, :].astype(F32),
                0.0,
            )

        for rdma in rdmas.values():
            rdma.wait_send()

    dma = pltpu.SemaphoreType.DMA
    return pl.pallas_call(
        body,
        out_shape=jax.ShapeDtypeStruct((m_chunk, n), F32),
        in_specs=[
            pl.BlockSpec(memory_space=pltpu.VMEM),
            pl.BlockSpec(memory_space=pltpu.VMEM),
        ],
        out_specs=pl.BlockSpec(memory_space=pltpu.VMEM),
        scratch_shapes=[
            pltpu.VMEM((m, k), F32),
            pltpu.VMEM((m, A_N), F32),
            pltpu.VMEM((m, n - A_N), F32),
            pltpu.VMEM((NA, 2, 2 * blk, COLW), BF),
            pltpu.VMEM((NA, 2, blk, COLW), BF),
            pltpu.VMEM((NA, blk, COLW), F32),
            pltpu.VMEM((NA, 2, 2 * m_chunk, COLW), BF),
            pltpu.VMEM((NA, 2, m_chunk, COLW), BF),
            pltpu.VMEM((NB, 2, 2 * blk, COLW), BF),
            pltpu.VMEM((NB, 2, blk, COLW), BF),
            pltpu.VMEM((NB, blk, COLW), F32),
            pltpu.VMEM((NB, 2, 2 * m_chunk, COLW), BF),
            pltpu.VMEM((NB, 2, m_chunk, COLW), BF),
            dma((NA,)), dma((NA,)), dma((NA,)), dma((NA,)),
            dma((NA,)), dma((NA,)), dma((NA,)), dma((NA,)),
            dma((NB,)), dma((NB,)), dma((NB,)), dma((NB,)),
            dma((NB,)), dma((NB,)), dma((NB,)), dma((NB,)),
        ],
        compiler_params=pltpu.CompilerParams(collective_id=0),
    )(x, w_mat)
